# Initial kernel scaffold; baseline (speedup 1.0000x reference)
import jax
import jax.numpy as jnp
from jax.experimental import pallas as pl


def kernel(x, x_attr, y, y_attr, edge_index, batches_seen, W_m1, b_m1, W_m2, b_m2, W_u1, b_u1, W_u2, b_u2, W_u3, b_u3):
    raise NotImplementedError("write your pallas kernel here")


# speedup vs baseline: 2.6116x; 2.6116x over previous
import functools

import jax
import jax.numpy as jnp
from jax import lax
from jax.experimental import pallas as pl
from jax.experimental.pallas import tpu as pltpu
from jax.experimental.pallas import tpu_sc as plsc

N = 10000
E = 320000
D = 128

NC = 2
NS = 16
NW = NC * NS
EPW = E // NW
K = 80
NCHUNK = EPW // K

_F32 = jnp.float32


def _node_in_body(x_ref, w1a_ref, w1b_ref, b1_ref, p_ref, q_ref):
    xb = x_ref[...]
    p_ref[...] = jnp.dot(xb, w1a_ref[...], preferred_element_type=_F32) + b1_ref[...]
    q_ref[...] = jnp.dot(xb, w1b_ref[...], preferred_element_type=_F32)


_node_in = pl.pallas_call(
    _node_in_body,
    grid=(5,),
    in_specs=[
        pl.BlockSpec((2000, D), lambda i: (i, 0)),
        pl.BlockSpec((D, D), lambda i: (0, 0)),
        pl.BlockSpec((D, D), lambda i: (0, 0)),
        pl.BlockSpec((1, D), lambda i: (0, 0)),
    ],
    out_specs=[
        pl.BlockSpec((2000, D), lambda i: (i, 0)),
        pl.BlockSpec((2000, D), lambda i: (i, 0)),
    ],
    out_shape=[
        jax.ShapeDtypeStruct((N, D), _F32),
        jax.ShapeDtypeStruct((N, D), _F32),
    ],
)


def _edge_mlp_body(h_ref, w2_ref, b2_ref, m_ref):
    h = h_ref[...]
    m_ref[...] = jnp.maximum(
        jnp.dot(h, w2_ref[...], preferred_element_type=_F32) + b2_ref[...], 0.0)


_edge_mlp = pl.pallas_call(
    _edge_mlp_body,
    grid=(50,),
    in_specs=[
        pl.BlockSpec((6400, D), lambda i: (i, 0)),
        pl.BlockSpec((D, D), lambda i: (0, 0)),
        pl.BlockSpec((1, D), lambda i: (0, 0)),
    ],
    out_specs=pl.BlockSpec((6400, D), lambda i: (i, 0)),
    out_shape=jax.ShapeDtypeStruct((E, D), _F32),
)


def _update_body(xx_ref, a0_ref, a1_ref, wu1_ref, bu1_ref, wu2_ref, bu2_ref,
                 wu3_ref, bu3_ref, w1a_ref, w1b_ref, b1_ref,
                 out_ref, p_ref, q_ref):
    agg = a0_ref[...] + a1_ref[...]
    u = jnp.maximum(
        jnp.dot(agg, wu1_ref[...], preferred_element_type=_F32) + bu1_ref[...], 0.0)
    u = jnp.maximum(
        jnp.dot(u, wu2_ref[...], preferred_element_type=_F32) + bu2_ref[...], 0.0)
    u = jnp.dot(u, wu3_ref[...], preferred_element_type=_F32) + bu3_ref[...]
    o = xx_ref[...] + u
    out_ref[...] = o
    p_ref[...] = jnp.dot(o, w1a_ref[...], preferred_element_type=_F32) + b1_ref[...]
    q_ref[...] = jnp.dot(o, w1b_ref[...], preferred_element_type=_F32)


_update = pl.pallas_call(
    _update_body,
    grid=(5,),
    in_specs=[
        pl.BlockSpec((2000, D), lambda i: (i, 0)),
        pl.BlockSpec((2000, D), lambda i: (i, 0)),
        pl.BlockSpec((2000, D), lambda i: (i, 0)),
    ] + [pl.BlockSpec((D, D), lambda i: (0, 0)), pl.BlockSpec((1, D), lambda i: (0, 0))] * 3
      + [pl.BlockSpec((D, D), lambda i: (0, 0)),
         pl.BlockSpec((D, D), lambda i: (0, 0)),
         pl.BlockSpec((1, D), lambda i: (0, 0))],
    out_specs=[
        pl.BlockSpec((2000, D), lambda i: (i, 0)),
        pl.BlockSpec((2000, D), lambda i: (i, 0)),
        pl.BlockSpec((2000, D), lambda i: (i, 0)),
    ],
    out_shape=[
        jax.ShapeDtypeStruct((N, D), _F32),
        jax.ShapeDtypeStruct((N, D), _F32),
        jax.ShapeDtypeStruct((N, D), _F32),
    ],
)


_MESH = plsc.VectorSubcoreMesh(core_axis_name="c", subcore_axis_name="s")


@functools.partial(
    pl.kernel,
    out_type=jax.ShapeDtypeStruct((E, D), _F32),
    mesh=_MESH,
    scratch_types=[
        pltpu.VMEM((K,), jnp.int32),
        pltpu.VMEM((K,), jnp.int32),
        pltpu.VMEM((K, D), _F32),
        pltpu.VMEM((K, D), _F32),
        pltpu.SemaphoreType.DMA,
        pltpu.SemaphoreType.DMA,
    ],
)
def _sc_gather(p_hbm, q_hbm, dst_hbm, src_hbm, h_hbm,
               dsti, srci, prow, qrow, sem1, sem2):
    wid = lax.axis_index("s") * NC + lax.axis_index("c")
    base = wid * EPW

    def chunk(ci, carry):
        off = base + ci * K
        pltpu.sync_copy(dst_hbm.at[pl.ds(off, K)], dsti)
        pltpu.sync_copy(src_hbm.at[pl.ds(off, K)], srci)
        cp1 = pltpu.async_copy(p_hbm.at[dsti], prow, sem1)
        cp2 = pltpu.async_copy(q_hbm.at[srci], qrow, sem2)
        cp1.wait()
        cp2.wait()

        def row(i, c2):
            for c in range(D // 16):
                sl = pl.ds(c * 16, 16)
                prow[i, sl] = jnp.maximum(prow[i, sl] + qrow[i, sl], 0.0)
            return c2

        lax.fori_loop(0, K, row, 0)
        pltpu.sync_copy(prow, h_hbm.at[pl.ds(off, K)])
        return carry

    lax.fori_loop(0, NCHUNK, chunk, 0)


_CH = 200
_NCH = N // _CH


@functools.partial(
    pl.kernel,
    out_type=jax.ShapeDtypeStruct((NC, N, D), _F32),
    mesh=_MESH,
    scratch_types=[
        pltpu.VMEM((K,), jnp.int32),
        pltpu.VMEM((K, D), _F32),
        pltpu.VMEM((_CH, D), _F32),
        pltpu.VMEM_SHARED((N, D), _F32),
        pltpu.SemaphoreType.DMA,
    ],
)
def _sc_scatter(m_hbm, dst_hbm, out_hbm, idxv, mrow, zbuf, acc, sem):
    cid = lax.axis_index("c")
    sid = lax.axis_index("s")
    wid = sid * NC + cid
    base = wid * EPW

    def zrow(i, carry):
        for c in range(D // 16):
            zbuf[i, pl.ds(c * 16, 16)] = jnp.zeros((16,), _F32)
        return carry

    lax.fori_loop(0, _CH, zrow, 0)

    def zchunk(j, carry):
        cidx = j * NS + sid

        @pl.when(cidx < _NCH)
        def _():
            pltpu.sync_copy(zbuf, acc.at[pl.ds(cidx * _CH, _CH)])

        return carry

    lax.fori_loop(0, (_NCH + NS - 1) // NS, zchunk, 0)
    plsc.subcore_barrier()

    def chunk(ci, carry):
        off = base + ci * K
        pltpu.sync_copy(dst_hbm.at[pl.ds(off, K)], idxv)
        pltpu.sync_copy(m_hbm.at[pl.ds(off, K)], mrow)
        pltpu.sync_copy(mrow, acc.at[idxv], add=True)
        return carry

    lax.fori_loop(0, NCHUNK, chunk, 0)
    plsc.subcore_barrier()

    def wchunk(j, carry):
        cidx = j * NS + sid

        @pl.when(cidx < _NCH)
        def _():
            pltpu.sync_copy(acc.at[pl.ds(cidx * _CH, _CH)],
                            out_hbm.at[cid, pl.ds(cidx * _CH, _CH)])

        return carry

    lax.fori_loop(0, (_NCH + NS - 1) // NS, wchunk, 0)


def kernel(x, x_attr, y, y_attr, edge_index, batches_seen,
           W_m1, b_m1, W_m2, b_m2, W_u1, b_u1, W_u2, b_u2, W_u3, b_u3):
    src = edge_index[0]
    dst = edge_index[1]
    w1a = W_m1[:D]
    w1b = W_m1[D:]
    b1 = b_m1.reshape(1, D)
    b2 = b_m2.reshape(1, D)
    bu1 = b_u1.reshape(1, D)
    bu2 = b_u2.reshape(1, D)
    bu3 = b_u3.reshape(1, D)

    xx = x[0, -1]
    p, q = _node_in(xx, w1a, w1b, b1)

    outs = []
    for _ in range(4):
        h = _sc_gather(p, q, dst, src)
        m = _edge_mlp(h, W_m2, b2)
        aggp = _sc_scatter(m, dst)
        xx, p, q = _update(xx, aggp[0], aggp[1], W_u1, bu1, W_u2, bu2,
                           W_u3, bu3, w1a, w1b, b1)
        outs.append(xx)

    out = jnp.stack(outs, axis=0)
    return out[None]


# speedup vs baseline: 5.0223x; 1.9231x over previous
import functools

import jax
import jax.numpy as jnp
from jax import lax
from jax.experimental import pallas as pl
from jax.experimental.pallas import tpu as pltpu
from jax.experimental.pallas import tpu_sc as plsc

N = 10000
E = 320000
D = 128

NC = 2
NS = 16
NW = NC * NS
EPW = E // NW
K = 80
NCHUNK = EPW // K

_F32 = jnp.float32


def _node_in_body(x_ref, w1a_ref, w1b_ref, b1_ref, p_ref, q_ref):
    xb = x_ref[...]
    p_ref[...] = jnp.dot(xb, w1a_ref[...], preferred_element_type=_F32) + b1_ref[...]
    q_ref[...] = jnp.dot(xb, w1b_ref[...], preferred_element_type=_F32)


_node_in = pl.pallas_call(
    _node_in_body,
    grid=(5,),
    in_specs=[
        pl.BlockSpec((2000, D), lambda i: (i, 0)),
        pl.BlockSpec((D, D), lambda i: (0, 0)),
        pl.BlockSpec((D, D), lambda i: (0, 0)),
        pl.BlockSpec((1, D), lambda i: (0, 0)),
    ],
    out_specs=[
        pl.BlockSpec((2000, D), lambda i: (i, 0)),
        pl.BlockSpec((2000, D), lambda i: (i, 0)),
    ],
    out_shape=[
        jax.ShapeDtypeStruct((N, D), _F32),
        jax.ShapeDtypeStruct((N, D), _F32),
    ],
)


def _edge_mlp_body(h_ref, w2_ref, b2_ref, m_ref):
    h = h_ref[...]
    m_ref[...] = jnp.maximum(
        jnp.dot(h, w2_ref[...], preferred_element_type=_F32) + b2_ref[...], 0.0)


_edge_mlp = pl.pallas_call(
    _edge_mlp_body,
    grid=(50,),
    in_specs=[
        pl.BlockSpec((6400, D), lambda i: (i, 0)),
        pl.BlockSpec((D, D), lambda i: (0, 0)),
        pl.BlockSpec((1, D), lambda i: (0, 0)),
    ],
    out_specs=pl.BlockSpec((6400, D), lambda i: (i, 0)),
    out_shape=jax.ShapeDtypeStruct((E, D), _F32),
)


def _update_body(xx_ref, a0_ref, a1_ref, wu1_ref, bu1_ref, wu2_ref, bu2_ref,
                 wu3_ref, bu3_ref, w1a_ref, w1b_ref, b1_ref,
                 out_ref, p_ref, q_ref):
    agg = a0_ref[...] + a1_ref[...]
    u = jnp.maximum(
        jnp.dot(agg, wu1_ref[...], preferred_element_type=_F32) + bu1_ref[...], 0.0)
    u = jnp.maximum(
        jnp.dot(u, wu2_ref[...], preferred_element_type=_F32) + bu2_ref[...], 0.0)
    u = jnp.dot(u, wu3_ref[...], preferred_element_type=_F32) + bu3_ref[...]
    o = xx_ref[...] + u
    out_ref[...] = o
    p_ref[...] = jnp.dot(o, w1a_ref[...], preferred_element_type=_F32) + b1_ref[...]
    q_ref[...] = jnp.dot(o, w1b_ref[...], preferred_element_type=_F32)


_update = pl.pallas_call(
    _update_body,
    grid=(5,),
    in_specs=[
        pl.BlockSpec((2000, D), lambda i: (i, 0)),
        pl.BlockSpec((2000, D), lambda i: (i, 0)),
        pl.BlockSpec((2000, D), lambda i: (i, 0)),
    ] + [pl.BlockSpec((D, D), lambda i: (0, 0)), pl.BlockSpec((1, D), lambda i: (0, 0))] * 3
      + [pl.BlockSpec((D, D), lambda i: (0, 0)),
         pl.BlockSpec((D, D), lambda i: (0, 0)),
         pl.BlockSpec((1, D), lambda i: (0, 0))],
    out_specs=[
        pl.BlockSpec((2000, D), lambda i: (i, 0)),
        pl.BlockSpec((2000, D), lambda i: (i, 0)),
        pl.BlockSpec((2000, D), lambda i: (i, 0)),
    ],
    out_shape=[
        jax.ShapeDtypeStruct((N, D), _F32),
        jax.ShapeDtypeStruct((N, D), _F32),
        jax.ShapeDtypeStruct((N, D), _F32),
    ],
)


_MESH = plsc.VectorSubcoreMesh(core_axis_name="c", subcore_axis_name="s")


@functools.partial(
    pl.kernel,
    out_type=jax.ShapeDtypeStruct((E, D), _F32),
    mesh=_MESH,
    scratch_types=[
        pltpu.VMEM((NCHUNK, K), jnp.int32),
        pltpu.VMEM((NCHUNK, K), jnp.int32),
        pltpu.VMEM((3, K, D), _F32),
        pltpu.VMEM((3, K, D), _F32),
        pltpu.VMEM((2, K, D), _F32),
        pltpu.SemaphoreType.DMA,
        pltpu.SemaphoreType.DMA,
        pltpu.SemaphoreType.DMA,
        pltpu.SemaphoreType.DMA,
        pltpu.SemaphoreType.DMA,
    ],
)
def _sc_gather(p_hbm, q_hbm, dst_hbm, src_hbm, h_hbm,
               di, si, pb, qb, hb, gs0, gs1, gs2, ws0, ws1):
    wid = lax.axis_index("s") * NC + lax.axis_index("c")
    base = wid * EPW
    gs = (gs0, gs1, gs2)
    ws = (ws0, ws1)

    pltpu.sync_copy(dst_hbm.at[wid], di)
    pltpu.sync_copy(src_hbm.at[wid], si)

    def issue(ci, b3):
        pltpu.async_copy(p_hbm.at[di.at[ci]], pb.at[b3], gs[b3])
        pltpu.async_copy(q_hbm.at[si.at[ci]], qb.at[b3], gs[b3])

    def gwait(b3):
        pltpu.make_async_copy(p_hbm.at[di.at[0]], pb.at[b3], gs[b3]).wait()
        pltpu.make_async_copy(q_hbm.at[si.at[0]], qb.at[b3], gs[b3]).wait()

    def wbwait(b2):
        pltpu.make_async_copy(hb.at[b2], h_hbm.at[pl.ds(0, K)], ws[b2]).wait()

    def do_chunk(ci, b3, b2, guard_wb, do_prefetch):
        gwait(b3)
        if guard_wb:
            @pl.when(ci >= 2)
            def _():
                wbwait(b2)
        else:
            wbwait(b2)

        def row(i, c2):
            for c in range(D // 16):
                sl = pl.ds(c * 16, 16)
                hb[b2, i, sl] = jnp.maximum(pb[b3, i, sl] + qb[b3, i, sl], 0.0)
            return c2

        lax.fori_loop(0, K, row, 0)
        pltpu.async_copy(hb.at[b2], h_hbm.at[pl.ds(base + ci * K, K)], ws[b2])
        if do_prefetch:
            issue(ci + 3, b3)

    for c0 in range(3):
        issue(c0, c0)

    def body(j, carry):
        ci0 = j * 6
        for u in range(6):
            do_chunk(ci0 + u, u % 3, u % 2, guard_wb=True, do_prefetch=True)
        return carry

    lax.fori_loop(0, 120 // 6, body, 0)

    for ci in range(120, NCHUNK):
        do_chunk(ci, ci % 3, ci % 2, guard_wb=False,
                 do_prefetch=(ci + 3 < NCHUNK))

    wbwait(1)
    wbwait(0)


_CH = 200
_NCH = N // _CH


@functools.partial(
    pl.kernel,
    out_type=jax.ShapeDtypeStruct((NC, N, D), _F32),
    mesh=_MESH,
    scratch_types=[
        pltpu.VMEM((NCHUNK, K), jnp.int32),
        pltpu.VMEM((2, K, D), _F32),
        pltpu.VMEM((K, D), _F32),
        pltpu.VMEM_SHARED((N, D), _F32),
        pltpu.SemaphoreType.DMA,
        pltpu.SemaphoreType.DMA,
    ],
)
def _sc_scatter(m_hbm, dst_hbm, out_hbm, di, mb, zbuf, acc, ms0, ms1):
    cid = lax.axis_index("c")
    sid = lax.axis_index("s")
    wid = sid * NC + cid
    base = wid * EPW
    ms = (ms0, ms1)

    pltpu.sync_copy(dst_hbm.at[wid], di)

    def zrow(i, carry):
        for c in range(D // 16):
            zbuf[i, pl.ds(c * 16, 16)] = jnp.zeros((16,), _F32)
        return carry

    lax.fori_loop(0, K, zrow, 0)

    def zchunk(j, carry):
        cidx = j * NS + sid

        @pl.when(cidx < NCHUNK)
        def _():
            pltpu.sync_copy(zbuf, acc.at[pl.ds(cidx * K, K)])

        return carry

    lax.fori_loop(0, (NCHUNK + NS - 1) // NS, zchunk, 0)
    plsc.subcore_barrier()

    def issue(ci, b2):
        pltpu.async_copy(m_hbm.at[pl.ds(base + ci * K, K)], mb.at[b2], ms[b2])

    def do_chunk(ci, b2, do_prefetch):
        pltpu.make_async_copy(m_hbm.at[pl.ds(0, K)], mb.at[b2], ms[b2]).wait()
        pltpu.sync_copy(mb.at[b2], acc.at[di.at[ci]], add=True)
        if do_prefetch:
            issue(ci + 2, b2)

    for c0 in range(2):
        issue(c0, c0)

    def body(j, carry):
        ci0 = j * 2
        for u in range(2):
            ci = ci0 + u

            do_chunk(ci, u, do_prefetch=False)

            @pl.when(ci + 2 < NCHUNK)
            def _():
                issue(ci + 2, u)

        return carry

    lax.fori_loop(0, NCHUNK // 2, body, 0)

    for ci in range((NCHUNK // 2) * 2, NCHUNK):
        do_chunk(ci, ci % 2, do_prefetch=False)

    plsc.subcore_barrier()

    def wchunk(j, carry):
        cidx = j * NS + sid

        @pl.when(cidx < _NCH)
        def _():
            pltpu.sync_copy(acc.at[pl.ds(cidx * _CH, _CH)],
                            out_hbm.at[cid, pl.ds(cidx * _CH, _CH)])

        return carry

    lax.fori_loop(0, (_NCH + NS - 1) // NS, wchunk, 0)


def kernel(x, x_attr, y, y_attr, edge_index, batches_seen,
           W_m1, b_m1, W_m2, b_m2, W_u1, b_u1, W_u2, b_u2, W_u3, b_u3):
    src = edge_index[0].reshape(NW, NCHUNK, K)
    dst = edge_index[1].reshape(NW, NCHUNK, K)
    w1a = W_m1[:D]
    w1b = W_m1[D:]
    b1 = b_m1.reshape(1, D)
    b2 = b_m2.reshape(1, D)
    bu1 = b_u1.reshape(1, D)
    bu2 = b_u2.reshape(1, D)
    bu3 = b_u3.reshape(1, D)

    xx = x[0, -1]
    p, q = _node_in(xx, w1a, w1b, b1)

    outs = []
    for _ in range(4):
        h = _sc_gather(p, q, dst, src)
        m = _edge_mlp(h, W_m2, b2)
        aggp = _sc_scatter(m, dst)
        xx, p, q = _update(xx, aggp[0], aggp[1], W_u1, bu1, W_u2, bu2,
                           W_u3, bu3, w1a, w1b, b1)
        outs.append(xx)

    out = jnp.stack(outs, axis=0)
    return out[None]
